# Initial kernel scaffold; baseline (speedup 1.0000x reference)
import jax
import jax.numpy as jnp
from jax.experimental import pallas as pl


def kernel(subtokens, W):
    raise NotImplementedError("write your pallas kernel here")


# speedup vs baseline: 6.0235x; 6.0235x over previous
import jax
import jax.numpy as jnp
from jax import lax
from jax.experimental import pallas as pl
from jax.experimental.pallas import tpu as pltpu
from jax.experimental.pallas import tpu_sc as plsc

VOCAB = 100000
EMBED = 64
BATCH = 1024
SEQ = 50
SUB = 8
N_TOK = BATCH * SEQ
NUM_WORKERS = 32
TOK_PER_WORKER = N_TOK // NUM_WORKERS
CHUNK = 80
NUM_CHUNKS = TOK_PER_WORKER // CHUNK
LANES = 16


def _sc_body(idsT_hbm, w_hbm, out_hbm, idx_v, rows_v, out_v, scale_v, sem):
    num_cores = 2
    wid = lax.axis_index("s") * num_cores + lax.axis_index("c")
    base = wid * TOK_PER_WORKER

    def chunk_body(ci, carry):
        tb = base + ci * CHUNK
        for s in range(SUB):
            pltpu.sync_copy(
                idsT_hbm.at[pl.ds(s * N_TOK + tb, CHUNK)], idx_v.at[s]
            )
        copies = [
            pltpu.async_copy(w_hbm.at[idx_v.at[s]], rows_v.at[s], sem)
            for s in range(SUB)
        ]
        for g in range(CHUNK // LANES):
            cnt = jnp.zeros((LANES,), jnp.float32)
            for s in range(SUB):
                ids16 = idx_v[s, pl.ds(g * LANES, LANES)]
                cnt = cnt + jnp.where(
                    ids16 != 0,
                    jnp.float32(1.0),
                    jnp.float32(0.0),
                )
            scale_v[pl.ds(g * LANES, LANES)] = 1.0 / (cnt + 1e-9)
        for c in copies:
            c.wait()

        def tok_body(t, inner):
            sc = scale_v[pl.ds(t, LANES)][0]
            for f in range(EMBED // LANES):
                acc = rows_v[0, t, pl.ds(f * LANES, LANES)]
                for s in range(1, SUB):
                    acc = acc + rows_v[s, t, pl.ds(f * LANES, LANES)]
                out_v[t, pl.ds(f * LANES, LANES)] = acc * sc
            return inner

        lax.fori_loop(0, CHUNK, tok_body, 0)
        pltpu.sync_copy(out_v, out_hbm.at[pl.ds(tb, CHUNK)])
        return carry

    lax.fori_loop(0, NUM_CHUNKS, chunk_body, 0)


_mesh = plsc.VectorSubcoreMesh(core_axis_name="c", subcore_axis_name="s")

_sc_call = pl.kernel(
    _sc_body,
    out_type=jax.ShapeDtypeStruct((N_TOK, EMBED), jnp.float32),
    mesh=_mesh,
    scratch_types=[
        pltpu.VMEM((SUB, CHUNK), jnp.int32),
        pltpu.VMEM((SUB, CHUNK, EMBED), jnp.float32),
        pltpu.VMEM((CHUNK, EMBED), jnp.float32),
        pltpu.VMEM((CHUNK + LANES,), jnp.float32),
        pltpu.SemaphoreType.DMA,
    ],
    compiler_params=pltpu.CompilerParams(use_tc_tiling_on_sc=False),
)


def kernel(subtokens, W):
    ids = subtokens.reshape(N_TOK, SUB).astype(jnp.int32)
    idsT = ids.T.reshape(-1)
    out = _sc_call(idsT, W)
    return out.reshape(BATCH, SEQ, EMBED)


# speedup vs baseline: 9.5522x; 1.5858x over previous
import jax
import jax.numpy as jnp
from jax import lax
from jax.experimental import pallas as pl
from jax.experimental.pallas import tpu as pltpu
from jax.experimental.pallas import tpu_sc as plsc

VOCAB = 100000
EMBED = 64
BATCH = 1024
SEQ = 50
SUB = 8
N_TOK = BATCH * SEQ
NUM_WORKERS = 32
TOK_PER_WORKER = N_TOK // NUM_WORKERS
CHUNK = 80
NUM_CHUNKS = TOK_PER_WORKER // CHUNK
LANES = 16


def _sc_body(ids_hbm, w_hbm, out_hbm, idx_v, rows_v, out_v, scale_v,
             sem_g0, sem_g1, sem_i0, sem_i1, sem_o0, sem_o1):
    sem_g = (sem_g0, sem_g1)
    sem_i = (sem_i0, sem_i1)
    sem_o = (sem_o0, sem_o1)
    num_cores = 2
    wid = lax.axis_index("s") * num_cores + lax.axis_index("c")
    blk0 = wid * NUM_CHUNKS

    pending_i = {}
    pending_g = {}
    pending_o = {}

    def fire_idx(ci):
        b = ci & 1
        off = (blk0 + ci) * (CHUNK * SUB)
        pending_i[ci] = pltpu.async_copy(
            ids_hbm.at[pl.ds(off, CHUNK * SUB)], idx_v.at[b], sem_i[b]
        )

    def fire_gathers(ci):
        b = ci & 1
        pending_g[ci] = [
            pltpu.async_copy(
                w_hbm.at[idx_v.at[b, pl.ds(s * CHUNK, CHUNK)]],
                rows_v.at[b, s],
                sem_g[b],
            )
            for s in range(SUB)
        ]

    def compute_scale(ci):
        b = ci & 1
        for g in range(CHUNK // LANES):
            cnt = jnp.zeros((LANES,), jnp.float32)
            for s in range(SUB):
                ids16 = idx_v[b, pl.ds(s * CHUNK + g * LANES, LANES)]
                cnt = cnt + jnp.where(
                    ids16 != 0, jnp.float32(1.0), jnp.float32(0.0)
                )
            scale_v[b, pl.ds(g * LANES, LANES)] = 1.0 / (cnt + 1e-9)

    def compute_chunk(ci):
        b = ci & 1

        def tok_body(t, inner):
            sc = scale_v[b, pl.ds(t, LANES)][0]
            for f in range(EMBED // LANES):
                acc = rows_v[b, 0, t, pl.ds(f * LANES, LANES)]
                for s in range(1, SUB):
                    acc = acc + rows_v[b, s, t, pl.ds(f * LANES, LANES)]
                out_v[b, t, pl.ds(f * LANES, LANES)] = acc * sc
            return inner

        lax.fori_loop(0, CHUNK, tok_body, 0)

    def fire_out(ci):
        b = ci & 1
        tb = (blk0 + ci) * CHUNK
        pending_o[ci] = pltpu.async_copy(
            out_v.at[b], out_hbm.at[pl.ds(tb, CHUNK)], sem_o[b]
        )

    fire_idx(0)
    pending_i.pop(0).wait()
    fire_gathers(0)
    compute_scale(0)
    fire_idx(1)

    for ci in range(NUM_CHUNKS):
        if ci + 1 < NUM_CHUNKS:
            pending_i.pop(ci + 1).wait()
            fire_gathers(ci + 1)
            compute_scale(ci + 1)
        for c in pending_g.pop(ci):
            c.wait()
        if ci + 2 < NUM_CHUNKS:
            fire_idx(ci + 2)
        if ci - 2 in pending_o:
            pending_o.pop(ci - 2).wait()
        compute_chunk(ci)
        fire_out(ci)

    pending_o.pop(NUM_CHUNKS - 2).wait()
    pending_o.pop(NUM_CHUNKS - 1).wait()


_mesh = plsc.VectorSubcoreMesh(core_axis_name="c", subcore_axis_name="s")

_sc_call = pl.kernel(
    _sc_body,
    out_type=jax.ShapeDtypeStruct((N_TOK, EMBED), jnp.float32),
    mesh=_mesh,
    scratch_types=[
        pltpu.VMEM((2, SUB * CHUNK), jnp.int32),
        pltpu.VMEM((2, SUB, CHUNK, EMBED), jnp.float32),
        pltpu.VMEM((2, CHUNK, EMBED), jnp.float32),
        pltpu.VMEM((2, CHUNK + LANES), jnp.float32),
        pltpu.SemaphoreType.DMA,
        pltpu.SemaphoreType.DMA,
        pltpu.SemaphoreType.DMA,
        pltpu.SemaphoreType.DMA,
        pltpu.SemaphoreType.DMA,
        pltpu.SemaphoreType.DMA,
    ],
    compiler_params=pltpu.CompilerParams(use_tc_tiling_on_sc=False),
)


def kernel(subtokens, W):
    ids = subtokens.reshape(N_TOK, SUB).astype(jnp.int32)
    ids_blocked = (
        ids.reshape(NUM_WORKERS, NUM_CHUNKS, CHUNK, SUB)
        .transpose(0, 1, 3, 2)
        .reshape(-1)
    )
    out = _sc_call(ids_blocked, W)
    return out.reshape(BATCH, SEQ, EMBED)
